# baseline (speedup 1.0000x reference)
import functools

import jax
import jax.numpy as jnp
from jax import lax
from jax.experimental import pallas as pl
from jax.experimental.pallas import tpu as pltpu
from jax.experimental.pallas import tpu_sc as plsc

_N_E = 8192
_E_DIM = 256
_BETA = 0.25
_TM = 512
_TE = 2048


def _dist_body(x_ref, cbt2_ref, idx_ref, cbn_ref):
    @pl.when(pl.program_id(0) == 0)
    def _():
        c = cbt2_ref[...]
        cbn_ref[...] = 0.25 * jnp.sum(c * c, axis=0, keepdims=True)

    x = x_ref[...]
    rn = jnp.sum(x * x, axis=1, keepdims=True)
    mm2 = lax.dot_general(x, cbt2_ref[...], (((1,), (0,)), ((), ())),
                          preferred_element_type=jnp.float32)
    d = (rn + cbn_ref[...]) - mm2
    m = jnp.min(d, axis=1, keepdims=True)
    ids = lax.broadcasted_iota(jnp.int32, d.shape, 1)
    idx_ref[0, 0, :] = jnp.min(jnp.where(d == m, ids, _N_E), axis=1)


def _indices(x2d, cbt):
    nt = x2d.shape[0]
    ni = nt // _TM
    out = pl.pallas_call(
        _dist_body,
        grid=(ni,),
        in_specs=[
            pl.BlockSpec((_TM, _E_DIM), lambda i: (i, 0)),
            pl.BlockSpec((_E_DIM, _N_E), lambda i: (0, 0)),
        ],
        out_specs=pl.BlockSpec((1, 1, _TM), lambda i: (i, 0, 0)),
        out_shape=jax.ShapeDtypeStruct((ni, 1, _TM), jnp.int32),
        scratch_shapes=[pltpu.VMEM((1, _N_E), jnp.float32)],
    )(x2d, cbt)
    return out.reshape(nt)


def _gather_rows(codebook, idx_flat):
    info = plsc.get_sparse_core_info()
    nw = info.num_cores * info.num_subcores
    b = idx_flat.shape[0]
    b_per_w = b // nw
    ch = 128
    nch = b_per_w // ch
    mesh = plsc.VectorSubcoreMesh(core_axis_name="c", subcore_axis_name="s")

    @functools.partial(
        pl.kernel, mesh=mesh,
        out_type=jax.ShapeDtypeStruct((b, _E_DIM), jnp.float32),
        scratch_types=[
            pltpu.VMEM((ch,), jnp.int32),
            pltpu.VMEM((ch, _E_DIM), jnp.float32),
            pltpu.SemaphoreType.DMA,
        ],
    )
    def k(cb_hbm, idx_hbm, out_hbm, idx_v, rows_v, sem):
        wid = lax.axis_index("s") * info.num_cores + lax.axis_index("c")
        base = wid * b_per_w

        def body(c, carry):
            off = base + c * ch
            pltpu.sync_copy(idx_hbm.at[pl.ds(off, ch)], idx_v)
            pltpu.async_copy(cb_hbm.at[idx_v], rows_v, sem).wait()
            pltpu.sync_copy(rows_v, out_hbm.at[pl.ds(off, ch)])
            return carry

        lax.fori_loop(0, nch, body, 0)

    return k(codebook, idx_flat)


def _st_body(x_ref, xq_ref, out_ref, loss_ref):
    x = x_ref[...]
    dlt = xq_ref[...] - x
    out_ref[...] = x + dlt

    @pl.when(pl.program_id(0) == 0)
    def _():
        loss_ref[0, 0] = 0.0

    loss_ref[0, 0] += jnp.sum(dlt * dlt)


def _st_and_loss(x2d, xq2d):
    nt = x2d.shape[0]
    ni = nt // _TE
    return pl.pallas_call(
        _st_body,
        grid=(ni,),
        in_specs=[
            pl.BlockSpec((_TE, _E_DIM), lambda i: (i, 0)),
            pl.BlockSpec((_TE, _E_DIM), lambda i: (i, 0)),
        ],
        out_specs=[
            pl.BlockSpec((_TE, _E_DIM), lambda i: (i, 0)),
            pl.BlockSpec(memory_space=pltpu.SMEM),
        ],
        out_shape=[
            jax.ShapeDtypeStruct((nt, _E_DIM), jnp.float32),
            jax.ShapeDtypeStruct((1, 1), jnp.float32),
        ],
    )(x2d, xq2d)


def kernel(x, codebook):
    x2d = x.reshape(-1, _E_DIM)
    cbt2 = (codebook + codebook).T
    idx_flat = _indices(x2d, cbt2)
    xq2d = _gather_rows(codebook, idx_flat)
    x_q_st, loss_sum = _st_and_loss(x2d, xq2d)
    m = loss_sum[0, 0] / x2d.size
    loss = m + _BETA * m
    return (x_q_st.reshape(x.shape), loss,
            idx_flat.reshape(x.shape[:-1]))


# speedup vs baseline: 1.0706x; 1.0706x over previous
import functools

import jax
import jax.numpy as jnp
from jax import lax
from jax.experimental import pallas as pl
from jax.experimental.pallas import tpu as pltpu
from jax.experimental.pallas import tpu_sc as plsc

_N_E = 8192
_E_DIM = 256
_BETA = 0.25
_TM = 512
_TE = 2048


def _dist_body(x_ref, cbt_ref, idx_ref, cbn_ref):
    @pl.when(pl.program_id(0) == 0)
    def _():
        c = cbt_ref[...]
        cbn_ref[...] = jnp.sum(c * c, axis=0, keepdims=True)

    x = x_ref[...]
    rn = jnp.sum(x * x, axis=1, keepdims=True)
    mm = lax.dot_general(x, cbt_ref[...], (((1,), (0,)), ((), ())),
                         preferred_element_type=jnp.float32)
    d = (rn + cbn_ref[...]) - 2.0 * mm
    m = jnp.min(d, axis=1, keepdims=True)
    ids = lax.broadcasted_iota(jnp.int32, d.shape, 1)
    idx_ref[0, 0, :] = jnp.min(jnp.where(d == m, ids, _N_E), axis=1)


def _indices(x2d, cbt):
    nt = x2d.shape[0]
    ni = nt // _TM
    out = pl.pallas_call(
        _dist_body,
        grid=(ni,),
        in_specs=[
            pl.BlockSpec((_TM, _E_DIM), lambda i: (i, 0)),
            pl.BlockSpec((_E_DIM, _N_E), lambda i: (0, 0)),
        ],
        out_specs=pl.BlockSpec((1, 1, _TM), lambda i: (i, 0, 0)),
        out_shape=jax.ShapeDtypeStruct((ni, 1, _TM), jnp.int32),
        scratch_shapes=[pltpu.VMEM((1, _N_E), jnp.float32)],
    )(x2d, cbt)
    return out.reshape(nt)


def _gather_rows(codebook, idx_flat):
    info = plsc.get_sparse_core_info()
    nw = info.num_cores * info.num_subcores
    b = idx_flat.shape[0]
    b_per_w = b // nw
    ch = 128
    nch = b_per_w // ch
    mesh = plsc.VectorSubcoreMesh(core_axis_name="c", subcore_axis_name="s")

    @functools.partial(
        pl.kernel, mesh=mesh,
        out_type=jax.ShapeDtypeStruct((b, _E_DIM), jnp.float32),
        scratch_types=[
            pltpu.VMEM((ch,), jnp.int32),
            pltpu.VMEM((ch, _E_DIM), jnp.float32),
            pltpu.SemaphoreType.DMA,
        ],
    )
    def k(cb_hbm, idx_hbm, out_hbm, idx_v, rows_v, sem):
        wid = lax.axis_index("s") * info.num_cores + lax.axis_index("c")
        base = wid * b_per_w

        def body(c, carry):
            off = base + c * ch
            pltpu.sync_copy(idx_hbm.at[pl.ds(off, ch)], idx_v)
            pltpu.async_copy(cb_hbm.at[idx_v], rows_v, sem).wait()
            pltpu.sync_copy(rows_v, out_hbm.at[pl.ds(off, ch)])
            return carry

        lax.fori_loop(0, nch, body, 0)

    return k(codebook, idx_flat)


def _st_body(x_ref, xq_ref, out_ref, loss_ref):
    x = x_ref[...]
    dlt = xq_ref[...] - x
    out_ref[...] = x + dlt

    @pl.when(pl.program_id(0) == 0)
    def _():
        loss_ref[0, 0] = 0.0

    loss_ref[0, 0] += jnp.sum(dlt * dlt)


def _st_and_loss(x2d, xq2d):
    nt = x2d.shape[0]
    ni = nt // _TE
    return pl.pallas_call(
        _st_body,
        grid=(ni,),
        in_specs=[
            pl.BlockSpec((_TE, _E_DIM), lambda i: (i, 0)),
            pl.BlockSpec((_TE, _E_DIM), lambda i: (i, 0)),
        ],
        out_specs=[
            pl.BlockSpec((_TE, _E_DIM), lambda i: (i, 0)),
            pl.BlockSpec(memory_space=pltpu.SMEM),
        ],
        out_shape=[
            jax.ShapeDtypeStruct((nt, _E_DIM), jnp.float32),
            jax.ShapeDtypeStruct((1, 1), jnp.float32),
        ],
    )(x2d, xq2d)


def kernel(x, codebook):
    x2d = x.reshape(-1, _E_DIM)
    cbt = codebook.T
    nslc = 4
    ch = x2d.shape[0] // nslc
    idxs, xqs = [], []
    for s in range(nslc):
        idx_s = _indices(x2d[s * ch:(s + 1) * ch], cbt)
        idxs.append(idx_s)
        xqs.append(_gather_rows(codebook, idx_s))
    idx_flat = jnp.concatenate(idxs)
    xq2d = jnp.concatenate(xqs)
    x_q_st, loss_sum = _st_and_loss(x2d, xq2d)
    m = loss_sum[0, 0] / x2d.size
    loss = m + _BETA * m
    return (x_q_st.reshape(x.shape), loss,
            idx_flat.reshape(x.shape[:-1]))


# speedup vs baseline: 1.1986x; 1.1196x over previous
import functools

import jax
import jax.numpy as jnp
from jax import lax
from jax.experimental import pallas as pl
from jax.experimental.pallas import tpu as pltpu
from jax.experimental.pallas import tpu_sc as plsc

_N_E = 8192
_E_DIM = 256
_BETA = 0.25
_TM = 1024
_TE = 2048


def _dist_body(x_ref, cbt_ref, idx_ref, cbn_ref):
    @pl.when(pl.program_id(0) == 0)
    def _():
        c = cbt_ref[...]
        cbn_ref[...] = jnp.sum(c * c, axis=0, keepdims=True)

    x = x_ref[...]
    rn = jnp.sum(x * x, axis=1, keepdims=True)
    mm = lax.dot_general(x, cbt_ref[...], (((1,), (0,)), ((), ())),
                         preferred_element_type=jnp.float32)
    d = (rn + cbn_ref[...]) - 2.0 * mm
    m = jnp.min(d, axis=1, keepdims=True)
    ids = lax.broadcasted_iota(jnp.int32, d.shape, 1)
    idx_ref[0, 0, :] = jnp.min(jnp.where(d == m, ids, _N_E), axis=1)


def _indices(x2d, cbt):
    nt = x2d.shape[0]
    ni = nt // _TM
    out = pl.pallas_call(
        _dist_body,
        grid=(ni,),
        in_specs=[
            pl.BlockSpec((_TM, _E_DIM), lambda i: (i, 0)),
            pl.BlockSpec((_E_DIM, _N_E), lambda i: (0, 0)),
        ],
        out_specs=pl.BlockSpec((1, 1, _TM), lambda i: (i, 0, 0)),
        out_shape=jax.ShapeDtypeStruct((ni, 1, _TM), jnp.int32),
        scratch_shapes=[pltpu.VMEM((1, _N_E), jnp.float32)],
    )(x2d, cbt)
    return out.reshape(nt)


def _gather_rows(codebook, idx_flat):
    info = plsc.get_sparse_core_info()
    nw = info.num_cores * info.num_subcores
    b = idx_flat.shape[0]
    b_per_w = b // nw
    ch = 128
    nch = b_per_w // ch
    mesh = plsc.VectorSubcoreMesh(core_axis_name="c", subcore_axis_name="s")

    @functools.partial(
        pl.kernel, mesh=mesh,
        out_type=jax.ShapeDtypeStruct((b, _E_DIM), jnp.float32),
        scratch_types=[
            pltpu.VMEM((ch,), jnp.int32),
            pltpu.VMEM((ch, _E_DIM), jnp.float32),
            pltpu.SemaphoreType.DMA,
        ],
    )
    def k(cb_hbm, idx_hbm, out_hbm, idx_v, rows_v, sem):
        wid = lax.axis_index("s") * info.num_cores + lax.axis_index("c")
        base = wid * b_per_w

        def body(c, carry):
            off = base + c * ch
            pltpu.sync_copy(idx_hbm.at[pl.ds(off, ch)], idx_v)
            pltpu.async_copy(cb_hbm.at[idx_v], rows_v, sem).wait()
            pltpu.sync_copy(rows_v, out_hbm.at[pl.ds(off, ch)])
            return carry

        lax.fori_loop(0, nch, body, 0)

    return k(codebook, idx_flat)


def _st_body(x_ref, xq_ref, out_ref, loss_ref):
    x = x_ref[...]
    dlt = xq_ref[...] - x
    out_ref[...] = x + dlt

    @pl.when(pl.program_id(0) == 0)
    def _():
        loss_ref[0, 0] = 0.0

    loss_ref[0, 0] += jnp.sum(dlt * dlt)


def _st_and_loss(x2d, xq2d):
    nt = x2d.shape[0]
    ni = nt // _TE
    return pl.pallas_call(
        _st_body,
        grid=(ni,),
        in_specs=[
            pl.BlockSpec((_TE, _E_DIM), lambda i: (i, 0)),
            pl.BlockSpec((_TE, _E_DIM), lambda i: (i, 0)),
        ],
        out_specs=[
            pl.BlockSpec((_TE, _E_DIM), lambda i: (i, 0)),
            pl.BlockSpec(memory_space=pltpu.SMEM),
        ],
        out_shape=[
            jax.ShapeDtypeStruct((nt, _E_DIM), jnp.float32),
            jax.ShapeDtypeStruct((1, 1), jnp.float32),
        ],
    )(x2d, xq2d)


def kernel(x, codebook):
    x2d = x.reshape(-1, _E_DIM)
    cbt = codebook.T
    idx_flat = _indices(x2d, cbt)
    xq2d = _gather_rows(codebook, idx_flat)
    x_q_st, loss_sum = _st_and_loss(x2d, xq2d)
    m = loss_sum[0, 0] / x2d.size
    loss = m + _BETA * m
    return (x_q_st.reshape(x.shape), loss,
            idx_flat.reshape(x.shape[:-1]))


# speedup vs baseline: 1.2079x; 1.0078x over previous
import functools

import jax
import jax.numpy as jnp
from jax import lax
from jax.experimental import pallas as pl
from jax.experimental.pallas import tpu as pltpu
from jax.experimental.pallas import tpu_sc as plsc

_N_E = 8192
_E_DIM = 256
_BETA = 0.25
_TM = 1024
_TE = 2048


def _dist_body(x_ref, cbt_ref, idx_ref, cbn_ref):
    @pl.when(pl.program_id(0) == 0)
    def _():
        c = cbt_ref[...]
        cbn_ref[...] = jnp.sum(c * c, axis=0, keepdims=True)

    x = x_ref[...]
    rn = jnp.sum(x * x, axis=1, keepdims=True)
    mm = lax.dot_general(x, cbt_ref[...], (((1,), (0,)), ((), ())),
                         preferred_element_type=jnp.float32)
    d = (rn + cbn_ref[...]) - 2.0 * mm
    m = jnp.min(d, axis=1, keepdims=True)
    ids = lax.broadcasted_iota(jnp.int32, d.shape, 1)
    idx_ref[0, 0, :] = jnp.min(jnp.where(d == m, ids, _N_E), axis=1)


def _indices(x2d, cbt):
    nt = x2d.shape[0]
    ni = nt // _TM
    out = pl.pallas_call(
        _dist_body,
        grid=(ni,),
        in_specs=[
            pl.BlockSpec((_TM, _E_DIM), lambda i: (i, 0)),
            pl.BlockSpec((_E_DIM, _N_E), lambda i: (0, 0)),
        ],
        out_specs=pl.BlockSpec((1, 1, _TM), lambda i: (i, 0, 0)),
        out_shape=jax.ShapeDtypeStruct((ni, 1, _TM), jnp.int32),
        scratch_shapes=[pltpu.VMEM((1, _N_E), jnp.float32)],
    )(x2d, cbt)
    return out.reshape(nt)


def _gather_rows(codebook, idx_flat):
    info = plsc.get_sparse_core_info()
    nw = info.num_cores * info.num_subcores
    b = idx_flat.shape[0]
    b_per_w = b // nw
    ch = 128
    nch = b_per_w // ch
    idx2 = idx_flat.reshape(b // ch, ch)
    mesh = plsc.VectorSubcoreMesh(core_axis_name="c", subcore_axis_name="s")

    @functools.partial(
        pl.kernel, mesh=mesh,
        out_type=jax.ShapeDtypeStruct((b, _E_DIM), jnp.float32),
        scratch_types=[
            pltpu.VMEM((nch, ch), jnp.int32),
            pltpu.VMEM((2, ch, _E_DIM), jnp.float32),
            pltpu.SemaphoreType.DMA((2,)),
            pltpu.SemaphoreType.DMA((2,)),
        ],
    )
    def k(cb_hbm, idx_hbm, out_hbm, idx_v, rows_v, gsem, wsem):
        wid = lax.axis_index("s") * info.num_cores + lax.axis_index("c")
        base = wid * b_per_w
        pltpu.sync_copy(idx_hbm.at[pl.ds(wid * nch, nch)], idx_v)

        def gather(c):
            return pltpu.async_copy(
                cb_hbm.at[idx_v.at[c]], rows_v.at[c % 2], gsem.at[c % 2])

        def wait_gather(c):
            pltpu.make_async_copy(
                cb_hbm.at[idx_v.at[c]], rows_v.at[c % 2],
                gsem.at[c % 2]).wait()

        def start_writeback(c):
            pltpu.async_copy(
                rows_v.at[c % 2], out_hbm.at[pl.ds(base + c * ch, ch)],
                wsem.at[c % 2])

        def wait_writeback(c):
            pltpu.make_async_copy(
                rows_v.at[c % 2], out_hbm.at[pl.ds(base + c * ch, ch)],
                wsem.at[c % 2]).wait()

        gather(0)
        for c in range(nch):
            wait_gather(c)
            if c >= 1:
                wait_writeback(c - 1)
            if c + 1 < nch:
                gather(c + 1)
            start_writeback(c)
        wait_writeback(nch - 1)

    return k(codebook, idx2)


def _st_body(x_ref, xq_ref, out_ref, loss_ref):
    x = x_ref[...]
    dlt = xq_ref[...] - x
    out_ref[...] = x + dlt

    @pl.when(pl.program_id(0) == 0)
    def _():
        loss_ref[0, 0] = 0.0

    loss_ref[0, 0] += jnp.sum(dlt * dlt)


def _st_and_loss(x2d, xq2d):
    nt = x2d.shape[0]
    ni = nt // _TE
    return pl.pallas_call(
        _st_body,
        grid=(ni,),
        in_specs=[
            pl.BlockSpec((_TE, _E_DIM), lambda i: (i, 0)),
            pl.BlockSpec((_TE, _E_DIM), lambda i: (i, 0)),
        ],
        out_specs=[
            pl.BlockSpec((_TE, _E_DIM), lambda i: (i, 0)),
            pl.BlockSpec(memory_space=pltpu.SMEM),
        ],
        out_shape=[
            jax.ShapeDtypeStruct((nt, _E_DIM), jnp.float32),
            jax.ShapeDtypeStruct((1, 1), jnp.float32),
        ],
    )(x2d, xq2d)


def kernel(x, codebook):
    x2d = x.reshape(-1, _E_DIM)
    cbt = codebook.T
    idx_flat = _indices(x2d, cbt)
    xq2d = _gather_rows(codebook, idx_flat)
    x_q_st, loss_sum = _st_and_loss(x2d, xq2d)
    m = loss_sum[0, 0] / x2d.size
    loss = m + _BETA * m
    return (x_q_st.reshape(x.shape), loss,
            idx_flat.reshape(x.shape[:-1]))


# speedup vs baseline: 1.2090x; 1.0009x over previous
import functools

import jax
import jax.numpy as jnp
from jax import lax
from jax.experimental import pallas as pl
from jax.experimental.pallas import tpu as pltpu
from jax.experimental.pallas import tpu_sc as plsc

_N_E = 8192
_E_DIM = 256
_BETA = 0.25
_TM = 1024
_TE = 2048


def _dist_body(x_ref, cbt_ref, idx_ref, cbn_ref):
    @pl.when(pl.program_id(0) == 0)
    def _():
        c = cbt_ref[...]
        cbn_ref[...] = jnp.sum(c * c, axis=0, keepdims=True)

    x = x_ref[...]
    rn = jnp.sum(x * x, axis=1, keepdims=True)
    mm = lax.dot_general(x, cbt_ref[...], (((1,), (0,)), ((), ())),
                         preferred_element_type=jnp.float32)
    d = (rn + cbn_ref[...]) - 2.0 * mm
    m = jnp.min(d, axis=1, keepdims=True)
    ids = lax.broadcasted_iota(jnp.int32, d.shape, 1)
    idx_ref[...] = jnp.min(jnp.where(d == m, ids, _N_E), axis=1)


def _indices(x2d, cbt):
    nt = x2d.shape[0]
    ni = nt // _TM
    out = pl.pallas_call(
        _dist_body,
        grid=(ni,),
        in_specs=[
            pl.BlockSpec((_TM, _E_DIM), lambda i: (i, 0)),
            pl.BlockSpec((_E_DIM, _N_E), lambda i: (0, 0)),
        ],
        out_specs=pl.BlockSpec((_TM,), lambda i: (i,)),
        out_shape=jax.ShapeDtypeStruct((nt,), jnp.int32),
        scratch_shapes=[pltpu.VMEM((1, _N_E), jnp.float32)],
    )(x2d, cbt)
    return out


def _gather_rows(codebook, idx_flat):
    info = plsc.get_sparse_core_info()
    nw = info.num_cores * info.num_subcores
    b = idx_flat.shape[0]
    b_per_w = b // nw
    ch = 128
    nch = b_per_w // ch
    idx2 = idx_flat.reshape(b // ch, ch)
    mesh = plsc.VectorSubcoreMesh(core_axis_name="c", subcore_axis_name="s")

    @functools.partial(
        pl.kernel, mesh=mesh,
        out_type=jax.ShapeDtypeStruct((b, _E_DIM), jnp.float32),
        scratch_types=[
            pltpu.VMEM((nch, ch), jnp.int32),
            pltpu.VMEM((2, ch, _E_DIM), jnp.float32),
            pltpu.SemaphoreType.DMA((2,)),
            pltpu.SemaphoreType.DMA((2,)),
        ],
    )
    def k(cb_hbm, idx_hbm, out_hbm, idx_v, rows_v, gsem, wsem):
        wid = lax.axis_index("s") * info.num_cores + lax.axis_index("c")
        base = wid * b_per_w
        pltpu.sync_copy(idx_hbm.at[pl.ds(wid * nch, nch)], idx_v)

        def gather(c):
            return pltpu.async_copy(
                cb_hbm.at[idx_v.at[c]], rows_v.at[c % 2], gsem.at[c % 2])

        def wait_gather(c):
            pltpu.make_async_copy(
                cb_hbm.at[idx_v.at[c]], rows_v.at[c % 2],
                gsem.at[c % 2]).wait()

        def start_writeback(c):
            pltpu.async_copy(
                rows_v.at[c % 2], out_hbm.at[pl.ds(base + c * ch, ch)],
                wsem.at[c % 2])

        def wait_writeback(c):
            pltpu.make_async_copy(
                rows_v.at[c % 2], out_hbm.at[pl.ds(base + c * ch, ch)],
                wsem.at[c % 2]).wait()

        gather(0)
        for c in range(nch):
            wait_gather(c)
            if c >= 1:
                wait_writeback(c - 1)
            if c + 1 < nch:
                gather(c + 1)
            start_writeback(c)
        wait_writeback(nch - 1)

    return k(codebook, idx2)


def _st_body(x_ref, xq_ref, out_ref, loss_ref):
    x = x_ref[...]
    dlt = xq_ref[...] - x
    out_ref[...] = x + dlt

    @pl.when(pl.program_id(0) == 0)
    def _():
        loss_ref[0, 0] = 0.0

    loss_ref[0, 0] += jnp.sum(dlt * dlt)


def _st_and_loss(x2d, xq2d):
    nt = x2d.shape[0]
    ni = nt // _TE
    return pl.pallas_call(
        _st_body,
        grid=(ni,),
        in_specs=[
            pl.BlockSpec((_TE, _E_DIM), lambda i: (i, 0)),
            pl.BlockSpec((_TE, _E_DIM), lambda i: (i, 0)),
        ],
        out_specs=[
            pl.BlockSpec((_TE, _E_DIM), lambda i: (i, 0)),
            pl.BlockSpec(memory_space=pltpu.SMEM),
        ],
        out_shape=[
            jax.ShapeDtypeStruct((nt, _E_DIM), jnp.float32),
            jax.ShapeDtypeStruct((1, 1), jnp.float32),
        ],
    )(x2d, xq2d)


def kernel(x, codebook):
    x2d = x.reshape(-1, _E_DIM)
    cbt = codebook.T
    idx_flat = _indices(x2d, cbt)
    xq2d = _gather_rows(codebook, idx_flat)
    x_q_st, loss_sum = _st_and_loss(x2d, xq2d)
    m = loss_sum[0, 0] / x2d.size
    loss = m + _BETA * m
    return (x_q_st.reshape(x.shape), loss,
            idx_flat.reshape(x.shape[:-1]))


# speedup vs baseline: 1.2126x; 1.0030x over previous
import functools

import jax
import jax.numpy as jnp
from jax import lax
from jax.experimental import pallas as pl
from jax.experimental.pallas import tpu as pltpu
from jax.experimental.pallas import tpu_sc as plsc

_N_E = 8192
_E_DIM = 256
_BETA = 0.25
_TM = 1024
_TE = 4096


def _dist_body(x_ref, cbt_ref, idx_ref, cbn_ref):
    @pl.when(pl.program_id(0) == 0)
    def _():
        c = cbt_ref[...]
        cbn_ref[...] = jnp.sum(c * c, axis=0, keepdims=True)

    x = x_ref[...]
    rn = jnp.sum(x * x, axis=1, keepdims=True)
    mm = lax.dot_general(x, cbt_ref[...], (((1,), (0,)), ((), ())),
                         preferred_element_type=jnp.float32)
    d = (rn + cbn_ref[...]) - 2.0 * mm
    m = jnp.min(d, axis=1, keepdims=True)
    ids = lax.broadcasted_iota(jnp.int32, d.shape, 1)
    idx_ref[...] = jnp.min(jnp.where(d == m, ids, _N_E), axis=1)


def _indices(x2d, cbt):
    nt = x2d.shape[0]
    ni = nt // _TM
    out = pl.pallas_call(
        _dist_body,
        grid=(ni,),
        in_specs=[
            pl.BlockSpec((_TM, _E_DIM), lambda i: (i, 0)),
            pl.BlockSpec((_E_DIM, _N_E), lambda i: (0, 0)),
        ],
        out_specs=pl.BlockSpec((_TM,), lambda i: (i,)),
        out_shape=jax.ShapeDtypeStruct((nt,), jnp.int32),
        scratch_shapes=[pltpu.VMEM((1, _N_E), jnp.float32)],
    )(x2d, cbt)
    return out


def _gather_rows(codebook, idx_flat):
    info = plsc.get_sparse_core_info()
    nw = info.num_cores * info.num_subcores
    b = idx_flat.shape[0]
    b_per_w = b // nw
    ch = 128
    nch = b_per_w // ch
    idx2 = idx_flat.reshape(b // ch, ch)
    mesh = plsc.VectorSubcoreMesh(core_axis_name="c", subcore_axis_name="s")

    @functools.partial(
        pl.kernel, mesh=mesh,
        out_type=jax.ShapeDtypeStruct((b, _E_DIM), jnp.float32),
        scratch_types=[
            pltpu.VMEM((nch, ch), jnp.int32),
            pltpu.VMEM((2, ch, _E_DIM), jnp.float32),
            pltpu.SemaphoreType.DMA((2,)),
            pltpu.SemaphoreType.DMA((2,)),
        ],
    )
    def k(cb_hbm, idx_hbm, out_hbm, idx_v, rows_v, gsem, wsem):
        wid = lax.axis_index("s") * info.num_cores + lax.axis_index("c")
        base = wid * b_per_w
        pltpu.sync_copy(idx_hbm.at[pl.ds(wid * nch, nch)], idx_v)

        def gather(c):
            return pltpu.async_copy(
                cb_hbm.at[idx_v.at[c]], rows_v.at[c % 2], gsem.at[c % 2])

        def wait_gather(c):
            pltpu.make_async_copy(
                cb_hbm.at[idx_v.at[c]], rows_v.at[c % 2],
                gsem.at[c % 2]).wait()

        def start_writeback(c):
            pltpu.async_copy(
                rows_v.at[c % 2], out_hbm.at[pl.ds(base + c * ch, ch)],
                wsem.at[c % 2])

        def wait_writeback(c):
            pltpu.make_async_copy(
                rows_v.at[c % 2], out_hbm.at[pl.ds(base + c * ch, ch)],
                wsem.at[c % 2]).wait()

        gather(0)
        for c in range(nch):
            wait_gather(c)
            if c >= 1:
                wait_writeback(c - 1)
            if c + 1 < nch:
                gather(c + 1)
            start_writeback(c)
        wait_writeback(nch - 1)

    return k(codebook, idx2)


def _st_body(x_ref, xq_ref, out_ref, loss_ref):
    x = x_ref[...]
    dlt = xq_ref[...] - x
    out_ref[...] = x + dlt

    @pl.when(pl.program_id(0) == 0)
    def _():
        loss_ref[0, 0] = 0.0

    loss_ref[0, 0] += jnp.sum(dlt * dlt)


def _st_and_loss(x2d, xq2d):
    nt = x2d.shape[0]
    ni = nt // _TE
    return pl.pallas_call(
        _st_body,
        grid=(ni,),
        in_specs=[
            pl.BlockSpec((_TE, _E_DIM), lambda i: (i, 0)),
            pl.BlockSpec((_TE, _E_DIM), lambda i: (i, 0)),
        ],
        out_specs=[
            pl.BlockSpec((_TE, _E_DIM), lambda i: (i, 0)),
            pl.BlockSpec(memory_space=pltpu.SMEM),
        ],
        out_shape=[
            jax.ShapeDtypeStruct((nt, _E_DIM), jnp.float32),
            jax.ShapeDtypeStruct((1, 1), jnp.float32),
        ],
    )(x2d, xq2d)


def kernel(x, codebook):
    x2d = x.reshape(-1, _E_DIM)
    cbt = codebook.T
    idx_flat = _indices(x2d, cbt)
    xq2d = _gather_rows(codebook, idx_flat)
    x_q_st, loss_sum = _st_and_loss(x2d, xq2d)
    m = loss_sum[0, 0] / x2d.size
    loss = m + _BETA * m
    return (x_q_st.reshape(x.shape), loss,
            idx_flat.reshape(x.shape[:-1]))
